# baseline (speedup 1.0000x reference)
import functools

import jax
import jax.numpy as jnp
import numpy as np
from jax import lax
from jax.experimental import pallas as pl
from jax.experimental.pallas import tpu as pltpu
from jax.experimental.pallas import tpu_sc as plsc

B = 8
N = 4096
NPOINT = 1024
RADIUS = 0.4
NSAMPLE = 64
R2 = np.float32(RADIUS * RADIUS)
M = B * NPOINT * NSAMPLE
TILE = 512
GRID = M // TILE
EPS = 1e-5


def _fps_body(xs_ref, ys_ref, zs_ref, out_ref):
    nb, n = xs_ref.shape
    xs = xs_ref[:, :]
    ys = ys_ref[:, :]
    zs = zs_ref[:, :]
    iota = jax.lax.broadcasted_iota(jnp.int32, (nb, n), 1)

    def step(k, carry):
        dists, f = carry
        out_ref[pl.ds(k, 1), :] = jnp.transpose(f)
        sel = iota == f
        cx = jnp.sum(jnp.where(sel, xs, 0.0), axis=1, keepdims=True)
        cy = jnp.sum(jnp.where(sel, ys, 0.0), axis=1, keepdims=True)
        cz = jnp.sum(jnp.where(sel, zs, 0.0), axis=1, keepdims=True)
        dx = xs - cx
        dy = ys - cy
        dz = zs - cz
        d = dx * dx + dy * dy + dz * dz
        dists = jnp.minimum(dists, d)
        m = jnp.max(dists, axis=1, keepdims=True)
        fn = jnp.min(jnp.where(dists == m, iota, n), axis=1, keepdims=True)
        return dists, fn.astype(jnp.int32)

    dists0 = jnp.full((nb, n), 1e10, dtype=jnp.float32)
    f0 = jnp.zeros((nb, 1), dtype=jnp.int32)
    jax.lax.fori_loop(0, NPOINT, step, (dists0, f0))


def _fps_pallas(xyz):
    out = pl.pallas_call(
        _fps_body,
        out_shape=jax.ShapeDtypeStruct((NPOINT, B), jnp.int32),
    )(xyz[:, :, 0], xyz[:, :, 1], xyz[:, :, 2])
    return jnp.transpose(out)


def _ptconv_body(x_ref, w_ref, b_ref, o_ref):
    o_ref[0] = jnp.dot(x_ref[0], w_ref[:, :],
                       preferred_element_type=jnp.float32) + b_ref[:, :]


def _ptconv(p_in, w1pad, b1):
    return pl.pallas_call(
        _ptconv_body,
        grid=(B, N // TILE),
        in_specs=[
            pl.BlockSpec((1, TILE, 136), lambda b, i: (b, i, 0)),
            pl.BlockSpec((136, 128), lambda b, i: (0, 0)),
            pl.BlockSpec((1, 128), lambda b, i: (0, 0)),
        ],
        out_specs=pl.BlockSpec((1, TILE, 128), lambda b, i: (b, i, 0)),
        out_shape=jax.ShapeDtypeStruct((B, N, 128), jnp.float32),
    )(p_in, w1pad, b1.reshape(1, 128))


def _u_body(x_ref, w_ref, o_ref):
    o_ref[:, :] = jnp.dot(x_ref[:, :], w_ref[:, :],
                          preferred_element_type=jnp.float32)


def _u_mat(nx_pad, w1xpad):
    return pl.pallas_call(
        _u_body,
        grid=(B * NPOINT // TILE,),
        in_specs=[
            pl.BlockSpec((TILE, 8), lambda i: (i, 0)),
            pl.BlockSpec((8, 128), lambda i: (0, 0)),
        ],
        out_specs=pl.BlockSpec((TILE, 128), lambda i: (i, 0)),
        out_shape=jax.ShapeDtypeStruct((B * NPOINT, 128), jnp.float32),
    )(nx_pad, w1xpad)


def _stats1_body(g_ref, u_ref, sum_ref, sq_ref):
    i = pl.program_id(0)
    y1 = (g_ref[:, :].astype(jnp.float32).reshape(8, NSAMPLE, 128)
          - u_ref[:, :].reshape(8, 1, 128)).reshape(TILE, 128)
    ps = jnp.sum(y1.reshape(64, 8, 128), axis=0)
    pq = jnp.sum((y1 * y1).reshape(64, 8, 128), axis=0)

    @pl.when(i == 0)
    def _():
        sum_ref[:, :] = jnp.zeros_like(sum_ref)
        sq_ref[:, :] = jnp.zeros_like(sq_ref)

    sum_ref[:, :] += ps
    sq_ref[:, :] += pq


def _stats1(g, u):
    return pl.pallas_call(
        _stats1_body,
        grid=(GRID,),
        in_specs=[
            pl.BlockSpec((TILE, 128), lambda i: (i, 0)),
            pl.BlockSpec((8, 128), lambda i: (i, 0)),
        ],
        out_specs=[
            pl.BlockSpec((8, 128), lambda i: (0, 0)),
            pl.BlockSpec((8, 128), lambda i: (0, 0)),
        ],
        out_shape=[jax.ShapeDtypeStruct((8, 128), jnp.float32)] * 2,
    )(g, u)


def _affine(sum_, sq_, gamma, beta):
    mean = jnp.sum(sum_, axis=0, keepdims=True) / M
    var = jnp.sum(sq_, axis=0, keepdims=True) / M - mean * mean
    scale = gamma.reshape(1, -1) / jnp.sqrt(var + EPS)
    shift = beta.reshape(1, -1) - mean * scale
    return scale, shift


def _mlp2_body(g_ref, u_ref, sum_ref, sq_ref, ga_ref, be_ref, w2_ref,
               x1_ref, sum2_ref, sq2_ref):
    i = pl.program_id(0)
    scale, shift = _affine(sum_ref[:, :], sq_ref[:, :], ga_ref[:, :], be_ref[:, :])
    y1 = (g_ref[:, :].astype(jnp.float32).reshape(8, NSAMPLE, 128)
          - u_ref[:, :].reshape(8, 1, 128)).reshape(TILE, 128)
    x1 = jnp.maximum(y1 * scale + shift, 0.0)
    x1_ref[:, :] = x1
    y2 = jnp.dot(x1, w2_ref[:, :], preferred_element_type=jnp.float32)
    ps = jnp.sum(y2.reshape(64, 8, 256), axis=0)
    pq = jnp.sum((y2 * y2).reshape(64, 8, 256), axis=0)

    @pl.when(i == 0)
    def _():
        sum2_ref[:, :] = jnp.zeros_like(sum2_ref)
        sq2_ref[:, :] = jnp.zeros_like(sq2_ref)

    sum2_ref[:, :] += ps
    sq2_ref[:, :] += pq


def _mlp2(g, u, s1, q1, g1, beta1, w2t):
    return pl.pallas_call(
        _mlp2_body,
        grid=(GRID,),
        in_specs=[
            pl.BlockSpec((TILE, 128), lambda i: (i, 0)),
            pl.BlockSpec((8, 128), lambda i: (i, 0)),
            pl.BlockSpec((8, 128), lambda i: (0, 0)),
            pl.BlockSpec((8, 128), lambda i: (0, 0)),
            pl.BlockSpec((1, 128), lambda i: (0, 0)),
            pl.BlockSpec((1, 128), lambda i: (0, 0)),
            pl.BlockSpec((128, 256), lambda i: (0, 0)),
        ],
        out_specs=[
            pl.BlockSpec((TILE, 128), lambda i: (i, 0)),
            pl.BlockSpec((8, 256), lambda i: (0, 0)),
            pl.BlockSpec((8, 256), lambda i: (0, 0)),
        ],
        out_shape=[
            jax.ShapeDtypeStruct((M, 128), jnp.float32),
            jax.ShapeDtypeStruct((8, 256), jnp.float32),
            jax.ShapeDtypeStruct((8, 256), jnp.float32),
        ],
    )(g, u, s1, q1, g1.reshape(1, 128), beta1.reshape(1, 128), w2t)


def _mlp3_body(x1_ref, sum2_ref, sq2_ref, ga_ref, be_ref, w2_ref, w3_ref,
               mx_ref, mn_ref, sum3_ref, sq3_ref):
    i = pl.program_id(0)
    scale, shift = _affine(sum2_ref[:, :], sq2_ref[:, :], ga_ref[:, :], be_ref[:, :])
    y2 = jnp.dot(x1_ref[:, :], w2_ref[:, :], preferred_element_type=jnp.float32)
    x2 = jnp.maximum(y2 * scale + shift, 0.0)
    y3 = jnp.dot(x2, w3_ref[:, :], preferred_element_type=jnp.float32)
    y3g = y3.reshape(8, NSAMPLE, 256)
    mx_ref[:, :] = jnp.max(y3g, axis=1)
    mn_ref[:, :] = jnp.min(y3g, axis=1)
    ps = jnp.sum(y3.reshape(64, 8, 256), axis=0)
    pq = jnp.sum((y3 * y3).reshape(64, 8, 256), axis=0)

    @pl.when(i == 0)
    def _():
        sum3_ref[:, :] = jnp.zeros_like(sum3_ref)
        sq3_ref[:, :] = jnp.zeros_like(sq3_ref)

    sum3_ref[:, :] += ps
    sq3_ref[:, :] += pq


def _mlp3(x1, s2, q2, g2, beta2, w2t, w3t):
    return pl.pallas_call(
        _mlp3_body,
        grid=(GRID,),
        in_specs=[
            pl.BlockSpec((TILE, 128), lambda i: (i, 0)),
            pl.BlockSpec((8, 256), lambda i: (0, 0)),
            pl.BlockSpec((8, 256), lambda i: (0, 0)),
            pl.BlockSpec((1, 256), lambda i: (0, 0)),
            pl.BlockSpec((1, 256), lambda i: (0, 0)),
            pl.BlockSpec((128, 256), lambda i: (0, 0)),
            pl.BlockSpec((256, 256), lambda i: (0, 0)),
        ],
        out_specs=[
            pl.BlockSpec((8, 256), lambda i: (i, 0)),
            pl.BlockSpec((8, 256), lambda i: (i, 0)),
            pl.BlockSpec((8, 256), lambda i: (0, 0)),
            pl.BlockSpec((8, 256), lambda i: (0, 0)),
        ],
        out_shape=[
            jax.ShapeDtypeStruct((B * NPOINT, 256), jnp.float32),
            jax.ShapeDtypeStruct((B * NPOINT, 256), jnp.float32),
            jax.ShapeDtypeStruct((8, 256), jnp.float32),
            jax.ShapeDtypeStruct((8, 256), jnp.float32),
        ],
    )(x1, s2, q2, g2.reshape(1, 256), beta2.reshape(1, 256), w2t, w3t)


def _fin_body(mx_ref, mn_ref, sum3_ref, sq3_ref, ga_ref, be_ref, o_ref):
    scale, shift = _affine(sum3_ref[:, :], sq3_ref[:, :], ga_ref[:, :], be_ref[:, :])
    hi = jnp.maximum(mx_ref[:, :] * scale + shift, 0.0)
    lo = jnp.maximum(mn_ref[:, :] * scale + shift, 0.0)
    o_ref[:, :] = jnp.where(scale > 0.0, hi, lo)


def _finalize(mx, mn, s3, q3, g3, beta3):
    return pl.pallas_call(
        _fin_body,
        grid=(B * NPOINT // TILE,),
        in_specs=[
            pl.BlockSpec((TILE, 256), lambda i: (i, 0)),
            pl.BlockSpec((TILE, 256), lambda i: (i, 0)),
            pl.BlockSpec((8, 256), lambda i: (0, 0)),
            pl.BlockSpec((8, 256), lambda i: (0, 0)),
            pl.BlockSpec((1, 256), lambda i: (0, 0)),
            pl.BlockSpec((1, 256), lambda i: (0, 0)),
        ],
        out_specs=pl.BlockSpec((TILE, 256), lambda i: (i, 0)),
        out_shape=jax.ShapeDtypeStruct((B * NPOINT, 256), jnp.float32),
    )(mx, mn, s3, q3, g3.reshape(1, 256), beta3.reshape(1, 256))


_ROWS = NPOINT // 4
_NV = N // 16


def _bq_row(p, xs_v, ys_v, zs_v, nx_v, ny_v, nz_v, idx_v, base):
    cx = jnp.full((16,), nx_v[pl.ds(p, 16)][0], jnp.float32)
    cy = jnp.full((16,), ny_v[pl.ds(p, 16)][0], jnp.float32)
    cz = jnp.full((16,), nz_v[pl.ds(p, 16)][0], jnp.float32)
    pad = jnp.full((16,), base, jnp.int32)
    idx_v[pl.ds(0, 16)] = pad
    idx_v[pl.ds(16, 16)] = pad
    idx_v[pl.ds(32, 16)] = pad
    idx_v[pl.ds(48, 16)] = pad
    lanes = lax.iota(jnp.int32, 16)

    def inner(i, cnt):
        off = pl.multiple_of(i * 16, 16)
        xv = xs_v[pl.ds(off, 16)]
        yv = ys_v[pl.ds(off, 16)]
        zv = zs_v[pl.ds(off, 16)]
        dx = xv - cx
        dy = yv - cy
        dz = zv - cz
        d2 = dx * dx + dy * dy + dz * dz
        m = d2 < R2
        vidx = lanes + (off + base)
        rank = plsc.cumsum(jnp.where(m, jnp.int32(1), jnp.int32(0)))
        plsc.store_scatter(idx_v, [cnt + rank - 1], vidx, mask=m)
        return cnt + plsc.all_reduce_population_count(m)

    cnt_vec = lax.fori_loop(0, _NV, inner, jnp.zeros((16,), jnp.int32))
    idx_v[pl.ds(cnt_vec[0], 16)] = pad


_NS = 4


def _bq_gather_body(xs_h, ys_h, zs_h, nx_h, ny_h, nz_h, c1_h, g_h, *rest):
    xs_v, ys_v, zs_v, nx_v, ny_v, nz_v, idxw = rest[:7]
    idx64 = rest[7:7 + _NS]
    gba = rest[7 + _NS:7 + 2 * _NS]
    gbb = rest[7 + 2 * _NS:7 + 3 * _NS]
    sga = rest[7 + 3 * _NS:7 + 4 * _NS]
    sgb = rest[7 + 4 * _NS:7 + 5 * _NS]
    soa = rest[7 + 5 * _NS:7 + 6 * _NS]
    sob = rest[7 + 6 * _NS:7 + 7 * _NS]
    cid = lax.axis_index("c")
    sid = lax.axis_index("s")
    wid = sid * 2 + cid
    b = wid // 4
    pb = (wid % 4) * _ROWS
    gp0 = wid * _ROWS
    pltpu.sync_copy(xs_h.at[b], xs_v)
    pltpu.sync_copy(ys_h.at[b], ys_v)
    pltpu.sync_copy(zs_h.at[b], zs_v)
    pltpu.sync_copy(nx_h.at[b, pl.ds(pb, _ROWS)], nx_v.at[pl.ds(0, _ROWS)])
    pltpu.sync_copy(ny_h.at[b, pl.ds(pb, _ROWS)], ny_v.at[pl.ds(0, _ROWS)])
    pltpu.sync_copy(nz_h.at[b, pl.ds(pb, _ROWS)], nz_v.at[pl.ds(0, _ROWS)])
    base = b * N

    def compute_into(p, s):
        _bq_row(p, xs_v, ys_v, zs_v, nx_v, ny_v, nz_v, idxw, base)
        for j in range(4):
            idx64[s][pl.ds(16 * j, 16)] = idxw[pl.ds(16 * j, 16)]

    def gather_start(s, gb, sg):
        pltpu.async_copy(c1_h.at[idx64[s]], gb[s], sg[s])

    def gather_wait(s, gb, sg):
        pltpu.make_async_copy(c1_h.at[idx64[s]], gb[s], sg[s]).wait()

    def out_start(r, s, gb, so):
        pltpu.async_copy(gb[s], g_h.at[gp0 + r], so[s])

    def out_wait(s, gb, so):
        pltpu.make_async_copy(gb[s], g_h.at[0], so[s]).wait()

    for s in range(_NS):
        compute_into(s, s)
        gather_start(s, gba, sga)
    for s in range(_NS):
        gather_wait(s, gba, sga)
        out_start(s, s, gba, soa)
        compute_into(_NS + s, s)
        gather_start(s, gbb, sgb)

    def pairbody(ii, _):
        r1 = _NS * (2 * ii + 1)
        for s in range(_NS):
            gather_wait(s, gbb, sgb)
            out_wait(s, gba, soa)
            out_start(r1 + s, s, gbb, sob)
            compute_into(r1 + _NS + s, s)
            gather_start(s, gba, sga)
        r2 = r1 + _NS
        for s in range(_NS):
            gather_wait(s, gba, sga)
            out_wait(s, gbb, sob)
            out_start(r2 + s, s, gba, soa)
            compute_into(r2 + _NS + s, s)
            gather_start(s, gbb, sgb)
        return 0

    lax.fori_loop(0, (_ROWS - 2 * _NS) // (2 * _NS), pairbody, 0)
    for s in range(_NS):
        gather_wait(s, gbb, sgb)
        out_wait(s, gba, soa)
        out_start(_ROWS - _NS + s, s, gbb, sob)
    for s in range(_NS):
        out_wait(s, gbb, sob)


def _bq_gather(xyz, new_xyz, c1):
    mesh = plsc.VectorSubcoreMesh(core_axis_name="c", subcore_axis_name="s")
    scratch = [
        pltpu.VMEM((N,), jnp.float32),
        pltpu.VMEM((N,), jnp.float32),
        pltpu.VMEM((N,), jnp.float32),
        pltpu.VMEM((_ROWS + 16,), jnp.float32),
        pltpu.VMEM((_ROWS + 16,), jnp.float32),
        pltpu.VMEM((_ROWS + 16,), jnp.float32),
        pltpu.VMEM((N + 16,), jnp.int32),
    ]
    scratch += [pltpu.VMEM((NSAMPLE,), jnp.int32) for _ in range(_NS)]
    scratch += [pltpu.VMEM((NSAMPLE, 128), jnp.float32) for _ in range(2 * _NS)]
    scratch += [pltpu.SemaphoreType.DMA for _ in range(4 * _NS)]
    f = pl.kernel(
        _bq_gather_body,
        out_type=jax.ShapeDtypeStruct((B * NPOINT, NSAMPLE, 128), jnp.float32),
        mesh=mesh,
        compiler_params=pltpu.CompilerParams(needs_layout_passes=False),
        scratch_types=scratch,
    )
    return f(xyz[:, :, 0], xyz[:, :, 1], xyz[:, :, 2],
             new_xyz[:, :, 0], new_xyz[:, :, 1], new_xyz[:, :, 2],
             c1.reshape(B * N, 128))


def kernel(xyz, features, W1, b1, g1, beta1, W2, b2, g2, beta2, W3, b3, g3, beta3):
    fidx = _fps_pallas(xyz)
    new_xyz = jax.vmap(lambda p, i: p[i])(xyz, fidx)

    p_in = jnp.concatenate(
        [jnp.transpose(features, (0, 2, 1)), xyz,
         jnp.zeros((B, N, 5), jnp.float32)], axis=2)
    w1pad = jnp.concatenate(
        [jnp.transpose(W1[:, 3:131]), jnp.transpose(W1[:, 0:3]),
         jnp.zeros((5, 128), jnp.float32)], axis=0)
    c1 = _ptconv(p_in, w1pad, b1)

    nx_pad = jnp.concatenate(
        [new_xyz.reshape(B * NPOINT, 3),
         jnp.zeros((B * NPOINT, 5), jnp.float32)], axis=1)
    w1xpad = jnp.concatenate(
        [jnp.transpose(W1[:, 0:3]), jnp.zeros((5, 128), jnp.float32)], axis=0)
    u = _u_mat(nx_pad, w1xpad)

    g = _bq_gather(xyz, new_xyz, c1).reshape(M, 128)

    s1, q1 = _stats1(g, u)
    w2t = jnp.transpose(W2)
    w3t = jnp.transpose(W3)
    x1, s2, q2 = _mlp2(g, u, s1, q1, g1, beta1, w2t)
    mx, mn, s3, q3 = _mlp3(x1, s2, q2, g2, beta2, w2t, w3t)
    out = _finalize(mx, mn, s3, q3, g3, beta3)
    new_features = jnp.transpose(out.reshape(B, NPOINT, 256), (0, 2, 1))
    return new_xyz, new_features


# speedup vs baseline: 1.0065x; 1.0065x over previous
import functools

import jax
import jax.numpy as jnp
import numpy as np
from jax import lax
from jax.experimental import pallas as pl
from jax.experimental.pallas import tpu as pltpu
from jax.experimental.pallas import tpu_sc as plsc

B = 8
N = 4096
NPOINT = 1024
RADIUS = 0.4
NSAMPLE = 64
R2 = np.float32(RADIUS * RADIUS)
M = B * NPOINT * NSAMPLE
TILE = 512
GRID = M // TILE
EPS = 1e-5


def _fps_body(xs_ref, ys_ref, zs_ref, out_ref):
    nb, n = xs_ref.shape
    xs = xs_ref[:, :]
    ys = ys_ref[:, :]
    zs = zs_ref[:, :]
    iota = jax.lax.broadcasted_iota(jnp.int32, (nb, n), 1)

    def step(k, carry):
        dists, f = carry
        out_ref[pl.ds(k, 1), :] = jnp.transpose(f)
        sel = iota == f
        cx = jnp.sum(jnp.where(sel, xs, 0.0), axis=1, keepdims=True)
        cy = jnp.sum(jnp.where(sel, ys, 0.0), axis=1, keepdims=True)
        cz = jnp.sum(jnp.where(sel, zs, 0.0), axis=1, keepdims=True)
        dx = xs - cx
        dy = ys - cy
        dz = zs - cz
        d = dx * dx + dy * dy + dz * dz
        dists = jnp.minimum(dists, d)
        m = jnp.max(dists, axis=1, keepdims=True)
        fn = jnp.min(jnp.where(dists == m, iota, n), axis=1, keepdims=True)
        return dists, fn.astype(jnp.int32)

    dists0 = jnp.full((nb, n), 1e10, dtype=jnp.float32)
    f0 = jnp.zeros((nb, 1), dtype=jnp.int32)
    jax.lax.fori_loop(0, NPOINT, step, (dists0, f0))


def _fps_pallas(xyz):
    out = pl.pallas_call(
        _fps_body,
        out_shape=jax.ShapeDtypeStruct((NPOINT, B), jnp.int32),
    )(xyz[:, :, 0], xyz[:, :, 1], xyz[:, :, 2])
    return jnp.transpose(out)


def _ptconv_body(x_ref, w_ref, b_ref, o_ref):
    o_ref[0] = jnp.dot(x_ref[0], w_ref[:, :],
                       preferred_element_type=jnp.float32) + b_ref[:, :]


def _ptconv(p_in, w1pad, b1):
    return pl.pallas_call(
        _ptconv_body,
        grid=(B, N // TILE),
        in_specs=[
            pl.BlockSpec((1, TILE, 136), lambda b, i: (b, i, 0)),
            pl.BlockSpec((136, 128), lambda b, i: (0, 0)),
            pl.BlockSpec((1, 128), lambda b, i: (0, 0)),
        ],
        out_specs=pl.BlockSpec((1, TILE, 128), lambda b, i: (b, i, 0)),
        out_shape=jax.ShapeDtypeStruct((B, N, 128), jnp.float32),
    )(p_in, w1pad, b1.reshape(1, 128))


def _u_body(x_ref, w_ref, o_ref):
    o_ref[:, :] = jnp.dot(x_ref[:, :], w_ref[:, :],
                          preferred_element_type=jnp.float32)


def _u_mat(nx_pad, w1xpad):
    return pl.pallas_call(
        _u_body,
        grid=(B * NPOINT // TILE,),
        in_specs=[
            pl.BlockSpec((TILE, 8), lambda i: (i, 0)),
            pl.BlockSpec((8, 128), lambda i: (0, 0)),
        ],
        out_specs=pl.BlockSpec((TILE, 128), lambda i: (i, 0)),
        out_shape=jax.ShapeDtypeStruct((B * NPOINT, 128), jnp.float32),
    )(nx_pad, w1xpad)


def _stats1_body(g_ref, u_ref, sum_ref, sq_ref):
    i = pl.program_id(0)
    y1 = (g_ref[:, :].astype(jnp.float32).reshape(8, NSAMPLE, 128)
          - u_ref[:, :].reshape(8, 1, 128)).reshape(TILE, 128)
    ps = jnp.sum(y1.reshape(64, 8, 128), axis=0)
    pq = jnp.sum((y1 * y1).reshape(64, 8, 128), axis=0)

    @pl.when(i == 0)
    def _():
        sum_ref[:, :] = jnp.zeros_like(sum_ref)
        sq_ref[:, :] = jnp.zeros_like(sq_ref)

    sum_ref[:, :] += ps
    sq_ref[:, :] += pq


def _stats1(g, u):
    return pl.pallas_call(
        _stats1_body,
        grid=(GRID,),
        in_specs=[
            pl.BlockSpec((TILE, 128), lambda i: (i, 0)),
            pl.BlockSpec((8, 128), lambda i: (i, 0)),
        ],
        out_specs=[
            pl.BlockSpec((8, 128), lambda i: (0, 0)),
            pl.BlockSpec((8, 128), lambda i: (0, 0)),
        ],
        out_shape=[jax.ShapeDtypeStruct((8, 128), jnp.float32)] * 2,
    )(g, u)


def _affine(sum_, sq_, gamma, beta):
    mean = jnp.sum(sum_, axis=0, keepdims=True) / M
    var = jnp.sum(sq_, axis=0, keepdims=True) / M - mean * mean
    scale = gamma.reshape(1, -1) / jnp.sqrt(var + EPS)
    shift = beta.reshape(1, -1) - mean * scale
    return scale, shift


def _mlp2_body(g_ref, u_ref, sum_ref, sq_ref, ga_ref, be_ref, w2_ref,
               x1_ref, sum2_ref, sq2_ref):
    i = pl.program_id(0)
    scale, shift = _affine(sum_ref[:, :], sq_ref[:, :], ga_ref[:, :], be_ref[:, :])
    y1 = (g_ref[:, :].astype(jnp.float32).reshape(8, NSAMPLE, 128)
          - u_ref[:, :].reshape(8, 1, 128)).reshape(TILE, 128)
    x1 = jnp.maximum(y1 * scale + shift, 0.0).astype(jnp.bfloat16)
    x1_ref[:, :] = x1
    y2 = jnp.dot(x1, w2_ref[:, :], preferred_element_type=jnp.float32)
    ps = jnp.sum(y2.reshape(64, 8, 256), axis=0)
    pq = jnp.sum((y2 * y2).reshape(64, 8, 256), axis=0)

    @pl.when(i == 0)
    def _():
        sum2_ref[:, :] = jnp.zeros_like(sum2_ref)
        sq2_ref[:, :] = jnp.zeros_like(sq2_ref)

    sum2_ref[:, :] += ps
    sq2_ref[:, :] += pq


def _mlp2(g, u, s1, q1, g1, beta1, w2t):
    return pl.pallas_call(
        _mlp2_body,
        grid=(GRID,),
        in_specs=[
            pl.BlockSpec((TILE, 128), lambda i: (i, 0)),
            pl.BlockSpec((8, 128), lambda i: (i, 0)),
            pl.BlockSpec((8, 128), lambda i: (0, 0)),
            pl.BlockSpec((8, 128), lambda i: (0, 0)),
            pl.BlockSpec((1, 128), lambda i: (0, 0)),
            pl.BlockSpec((1, 128), lambda i: (0, 0)),
            pl.BlockSpec((128, 256), lambda i: (0, 0)),
        ],
        out_specs=[
            pl.BlockSpec((TILE, 128), lambda i: (i, 0)),
            pl.BlockSpec((8, 256), lambda i: (0, 0)),
            pl.BlockSpec((8, 256), lambda i: (0, 0)),
        ],
        out_shape=[
            jax.ShapeDtypeStruct((M, 128), jnp.bfloat16),
            jax.ShapeDtypeStruct((8, 256), jnp.float32),
            jax.ShapeDtypeStruct((8, 256), jnp.float32),
        ],
    )(g, u, s1, q1, g1.reshape(1, 128), beta1.reshape(1, 128), w2t)


def _mlp3_body(x1_ref, sum2_ref, sq2_ref, ga_ref, be_ref, w2_ref, w3_ref,
               mx_ref, mn_ref, sum3_ref, sq3_ref):
    i = pl.program_id(0)
    scale, shift = _affine(sum2_ref[:, :], sq2_ref[:, :], ga_ref[:, :], be_ref[:, :])
    y2 = jnp.dot(x1_ref[:, :], w2_ref[:, :], preferred_element_type=jnp.float32)
    x2 = jnp.maximum(y2 * scale + shift, 0.0).astype(jnp.bfloat16)
    y3 = jnp.dot(x2, w3_ref[:, :], preferred_element_type=jnp.float32)
    y3g = y3.reshape(8, NSAMPLE, 256)
    mx_ref[:, :] = jnp.max(y3g, axis=1)
    mn_ref[:, :] = jnp.min(y3g, axis=1)
    ps = jnp.sum(y3.reshape(64, 8, 256), axis=0)
    pq = jnp.sum((y3 * y3).reshape(64, 8, 256), axis=0)

    @pl.when(i == 0)
    def _():
        sum3_ref[:, :] = jnp.zeros_like(sum3_ref)
        sq3_ref[:, :] = jnp.zeros_like(sq3_ref)

    sum3_ref[:, :] += ps
    sq3_ref[:, :] += pq


def _mlp3(x1, s2, q2, g2, beta2, w2t, w3t):
    return pl.pallas_call(
        _mlp3_body,
        grid=(GRID,),
        in_specs=[
            pl.BlockSpec((TILE, 128), lambda i: (i, 0)),
            pl.BlockSpec((8, 256), lambda i: (0, 0)),
            pl.BlockSpec((8, 256), lambda i: (0, 0)),
            pl.BlockSpec((1, 256), lambda i: (0, 0)),
            pl.BlockSpec((1, 256), lambda i: (0, 0)),
            pl.BlockSpec((128, 256), lambda i: (0, 0)),
            pl.BlockSpec((256, 256), lambda i: (0, 0)),
        ],
        out_specs=[
            pl.BlockSpec((8, 256), lambda i: (i, 0)),
            pl.BlockSpec((8, 256), lambda i: (i, 0)),
            pl.BlockSpec((8, 256), lambda i: (0, 0)),
            pl.BlockSpec((8, 256), lambda i: (0, 0)),
        ],
        out_shape=[
            jax.ShapeDtypeStruct((B * NPOINT, 256), jnp.float32),
            jax.ShapeDtypeStruct((B * NPOINT, 256), jnp.float32),
            jax.ShapeDtypeStruct((8, 256), jnp.float32),
            jax.ShapeDtypeStruct((8, 256), jnp.float32),
        ],
    )(x1, s2, q2, g2.reshape(1, 256), beta2.reshape(1, 256), w2t, w3t)


def _fin_body(mx_ref, mn_ref, sum3_ref, sq3_ref, ga_ref, be_ref, o_ref):
    scale, shift = _affine(sum3_ref[:, :], sq3_ref[:, :], ga_ref[:, :], be_ref[:, :])
    hi = jnp.maximum(mx_ref[:, :] * scale + shift, 0.0)
    lo = jnp.maximum(mn_ref[:, :] * scale + shift, 0.0)
    o_ref[:, :] = jnp.where(scale > 0.0, hi, lo)


def _finalize(mx, mn, s3, q3, g3, beta3):
    return pl.pallas_call(
        _fin_body,
        grid=(B * NPOINT // TILE,),
        in_specs=[
            pl.BlockSpec((TILE, 256), lambda i: (i, 0)),
            pl.BlockSpec((TILE, 256), lambda i: (i, 0)),
            pl.BlockSpec((8, 256), lambda i: (0, 0)),
            pl.BlockSpec((8, 256), lambda i: (0, 0)),
            pl.BlockSpec((1, 256), lambda i: (0, 0)),
            pl.BlockSpec((1, 256), lambda i: (0, 0)),
        ],
        out_specs=pl.BlockSpec((TILE, 256), lambda i: (i, 0)),
        out_shape=jax.ShapeDtypeStruct((B * NPOINT, 256), jnp.float32),
    )(mx, mn, s3, q3, g3.reshape(1, 256), beta3.reshape(1, 256))


_ROWS = NPOINT // 4
_NV = N // 16


def _bq_row(p, xs_v, ys_v, zs_v, nx_v, ny_v, nz_v, idx_v, base):
    cx = jnp.full((16,), nx_v[pl.ds(p, 16)][0], jnp.float32)
    cy = jnp.full((16,), ny_v[pl.ds(p, 16)][0], jnp.float32)
    cz = jnp.full((16,), nz_v[pl.ds(p, 16)][0], jnp.float32)
    pad = jnp.full((16,), base, jnp.int32)
    idx_v[pl.ds(0, 16)] = pad
    idx_v[pl.ds(16, 16)] = pad
    idx_v[pl.ds(32, 16)] = pad
    idx_v[pl.ds(48, 16)] = pad
    lanes = lax.iota(jnp.int32, 16)

    def inner(i, cnt):
        off = pl.multiple_of(i * 16, 16)
        xv = xs_v[pl.ds(off, 16)]
        yv = ys_v[pl.ds(off, 16)]
        zv = zs_v[pl.ds(off, 16)]
        dx = xv - cx
        dy = yv - cy
        dz = zv - cz
        d2 = dx * dx + dy * dy + dz * dz
        m = d2 < R2
        vidx = lanes + (off + base)
        rank = plsc.cumsum(jnp.where(m, jnp.int32(1), jnp.int32(0)))
        plsc.store_scatter(idx_v, [cnt + rank - 1], vidx, mask=m)
        return cnt + plsc.all_reduce_population_count(m)

    cnt_vec = lax.fori_loop(0, _NV, inner, jnp.zeros((16,), jnp.int32))
    idx_v[pl.ds(cnt_vec[0], 16)] = pad


_NS = 4


def _bq_gather_body(xs_h, ys_h, zs_h, nx_h, ny_h, nz_h, c1_h, g_h, *rest):
    xs_v, ys_v, zs_v, nx_v, ny_v, nz_v, idxw = rest[:7]
    idx64 = rest[7:7 + _NS]
    gba = rest[7 + _NS:7 + 2 * _NS]
    gbb = rest[7 + 2 * _NS:7 + 3 * _NS]
    sga = rest[7 + 3 * _NS:7 + 4 * _NS]
    sgb = rest[7 + 4 * _NS:7 + 5 * _NS]
    soa = rest[7 + 5 * _NS:7 + 6 * _NS]
    sob = rest[7 + 6 * _NS:7 + 7 * _NS]
    cid = lax.axis_index("c")
    sid = lax.axis_index("s")
    wid = sid * 2 + cid
    b = wid // 4
    pb = (wid % 4) * _ROWS
    gp0 = wid * _ROWS
    pltpu.sync_copy(xs_h.at[b], xs_v)
    pltpu.sync_copy(ys_h.at[b], ys_v)
    pltpu.sync_copy(zs_h.at[b], zs_v)
    pltpu.sync_copy(nx_h.at[b, pl.ds(pb, _ROWS)], nx_v.at[pl.ds(0, _ROWS)])
    pltpu.sync_copy(ny_h.at[b, pl.ds(pb, _ROWS)], ny_v.at[pl.ds(0, _ROWS)])
    pltpu.sync_copy(nz_h.at[b, pl.ds(pb, _ROWS)], nz_v.at[pl.ds(0, _ROWS)])
    base = b * N

    def compute_into(p, s):
        _bq_row(p, xs_v, ys_v, zs_v, nx_v, ny_v, nz_v, idxw, base)
        for j in range(4):
            idx64[s][pl.ds(16 * j, 16)] = idxw[pl.ds(16 * j, 16)]

    def gather_start(s, gb, sg):
        pltpu.async_copy(c1_h.at[idx64[s]], gb[s], sg[s])

    def gather_wait(s, gb, sg):
        pltpu.make_async_copy(c1_h.at[idx64[s]], gb[s], sg[s]).wait()

    def out_start(r, s, gb, so):
        pltpu.async_copy(gb[s], g_h.at[gp0 + r], so[s])

    def out_wait(s, gb, so):
        pltpu.make_async_copy(gb[s], g_h.at[0], so[s]).wait()

    for s in range(_NS):
        compute_into(s, s)
        gather_start(s, gba, sga)
    for s in range(_NS):
        gather_wait(s, gba, sga)
        out_start(s, s, gba, soa)
        compute_into(_NS + s, s)
        gather_start(s, gbb, sgb)

    def pairbody(ii, _):
        r1 = _NS * (2 * ii + 1)
        for s in range(_NS):
            gather_wait(s, gbb, sgb)
            out_wait(s, gba, soa)
            out_start(r1 + s, s, gbb, sob)
            compute_into(r1 + _NS + s, s)
            gather_start(s, gba, sga)
        r2 = r1 + _NS
        for s in range(_NS):
            gather_wait(s, gba, sga)
            out_wait(s, gbb, sob)
            out_start(r2 + s, s, gba, soa)
            compute_into(r2 + _NS + s, s)
            gather_start(s, gbb, sgb)
        return 0

    lax.fori_loop(0, (_ROWS - 2 * _NS) // (2 * _NS), pairbody, 0)
    for s in range(_NS):
        gather_wait(s, gbb, sgb)
        out_wait(s, gba, soa)
        out_start(_ROWS - _NS + s, s, gbb, sob)
    for s in range(_NS):
        out_wait(s, gbb, sob)


def _bq_gather(xyz, new_xyz, c1):
    mesh = plsc.VectorSubcoreMesh(core_axis_name="c", subcore_axis_name="s")
    scratch = [
        pltpu.VMEM((N,), jnp.float32),
        pltpu.VMEM((N,), jnp.float32),
        pltpu.VMEM((N,), jnp.float32),
        pltpu.VMEM((_ROWS + 16,), jnp.float32),
        pltpu.VMEM((_ROWS + 16,), jnp.float32),
        pltpu.VMEM((_ROWS + 16,), jnp.float32),
        pltpu.VMEM((N + 16,), jnp.int32),
    ]
    scratch += [pltpu.VMEM((NSAMPLE,), jnp.int32) for _ in range(_NS)]
    scratch += [pltpu.VMEM((NSAMPLE, 128), jnp.float32) for _ in range(2 * _NS)]
    scratch += [pltpu.SemaphoreType.DMA for _ in range(4 * _NS)]
    f = pl.kernel(
        _bq_gather_body,
        out_type=jax.ShapeDtypeStruct((B * NPOINT, NSAMPLE, 128), jnp.float32),
        mesh=mesh,
        compiler_params=pltpu.CompilerParams(needs_layout_passes=False),
        scratch_types=scratch,
    )
    return f(xyz[:, :, 0], xyz[:, :, 1], xyz[:, :, 2],
             new_xyz[:, :, 0], new_xyz[:, :, 1], new_xyz[:, :, 2],
             c1.reshape(B * N, 128))


def kernel(xyz, features, W1, b1, g1, beta1, W2, b2, g2, beta2, W3, b3, g3, beta3):
    fidx = _fps_pallas(xyz)
    new_xyz = jax.vmap(lambda p, i: p[i])(xyz, fidx)

    p_in = jnp.concatenate(
        [jnp.transpose(features, (0, 2, 1)), xyz,
         jnp.zeros((B, N, 5), jnp.float32)], axis=2)
    w1pad = jnp.concatenate(
        [jnp.transpose(W1[:, 3:131]), jnp.transpose(W1[:, 0:3]),
         jnp.zeros((5, 128), jnp.float32)], axis=0)
    c1 = _ptconv(p_in, w1pad, b1)

    nx_pad = jnp.concatenate(
        [new_xyz.reshape(B * NPOINT, 3),
         jnp.zeros((B * NPOINT, 5), jnp.float32)], axis=1)
    w1xpad = jnp.concatenate(
        [jnp.transpose(W1[:, 0:3]), jnp.zeros((5, 128), jnp.float32)], axis=0)
    u = _u_mat(nx_pad, w1xpad)

    g = _bq_gather(xyz, new_xyz, c1).reshape(M, 128)

    s1, q1 = _stats1(g, u)
    w2t = jnp.transpose(W2).astype(jnp.bfloat16)
    w3t = jnp.transpose(W3).astype(jnp.bfloat16)
    x1, s2, q2 = _mlp2(g, u, s1, q1, g1, beta1, w2t)
    mx, mn, s3, q3 = _mlp3(x1, s2, q2, g2, beta2, w2t, w3t)
    out = _finalize(mx, mn, s3, q3, g3, beta3)
    new_features = jnp.transpose(out.reshape(B, NPOINT, 256), (0, 2, 1))
    return new_xyz, new_features


# speedup vs baseline: 1.9037x; 1.8914x over previous
import functools

import jax
import jax.numpy as jnp
import numpy as np
from jax import lax
from jax.experimental import pallas as pl
from jax.experimental.pallas import tpu as pltpu
from jax.experimental.pallas import tpu_sc as plsc

B = 8
N = 4096
NPOINT = 1024
RADIUS = 0.4
NSAMPLE = 64
R2 = np.float32(RADIUS * RADIUS)
M = B * NPOINT * NSAMPLE
TILE = 512
GRID = M // TILE
EPS = 1e-5


def _fps_body(xs_ref, ys_ref, zs_ref, out_ref):
    nb, n = xs_ref.shape
    xs = xs_ref[:, :]
    ys = ys_ref[:, :]
    zs = zs_ref[:, :]
    iota = jax.lax.broadcasted_iota(jnp.int32, (nb, n), 1)

    def step(k, carry):
        dists, f = carry
        out_ref[pl.ds(k, 1), :] = jnp.transpose(f)
        sel = iota == f
        cx = jnp.sum(jnp.where(sel, xs, 0.0), axis=1, keepdims=True)
        cy = jnp.sum(jnp.where(sel, ys, 0.0), axis=1, keepdims=True)
        cz = jnp.sum(jnp.where(sel, zs, 0.0), axis=1, keepdims=True)
        dx = xs - cx
        dy = ys - cy
        dz = zs - cz
        d = dx * dx + dy * dy + dz * dz
        dists = jnp.minimum(dists, d)
        m = jnp.max(dists, axis=1, keepdims=True)
        fn = jnp.min(jnp.where(dists == m, iota, n), axis=1, keepdims=True)
        return dists, fn.astype(jnp.int32)

    dists0 = jnp.full((nb, n), 1e10, dtype=jnp.float32)
    f0 = jnp.zeros((nb, 1), dtype=jnp.int32)
    jax.lax.fori_loop(0, NPOINT, step, (dists0, f0))


def _fps_pallas(xyz):
    out = pl.pallas_call(
        _fps_body,
        out_shape=jax.ShapeDtypeStruct((NPOINT, B), jnp.int32),
    )(xyz[:, :, 0], xyz[:, :, 1], xyz[:, :, 2])
    return jnp.transpose(out)


def _ptconv_body(x_ref, w_ref, b_ref, o_ref):
    o_ref[0] = jnp.dot(x_ref[0], w_ref[:, :],
                       preferred_element_type=jnp.float32) + b_ref[:, :]


def _ptconv(p_in, w1pad, b1):
    return pl.pallas_call(
        _ptconv_body,
        grid=(B, N // TILE),
        in_specs=[
            pl.BlockSpec((1, TILE, 136), lambda b, i: (b, i, 0)),
            pl.BlockSpec((136, 128), lambda b, i: (0, 0)),
            pl.BlockSpec((1, 128), lambda b, i: (0, 0)),
        ],
        out_specs=pl.BlockSpec((1, TILE, 128), lambda b, i: (b, i, 0)),
        out_shape=jax.ShapeDtypeStruct((B, N, 128), jnp.float32),
    )(p_in, w1pad, b1.reshape(1, 128))


def _u_body(x_ref, w_ref, o_ref):
    o_ref[:, :] = jnp.dot(x_ref[:, :], w_ref[:, :],
                          preferred_element_type=jnp.float32)


def _u_mat(nx_pad, w1xpad):
    return pl.pallas_call(
        _u_body,
        grid=(B * NPOINT // TILE,),
        in_specs=[
            pl.BlockSpec((TILE, 8), lambda i: (i, 0)),
            pl.BlockSpec((8, 128), lambda i: (0, 0)),
        ],
        out_specs=pl.BlockSpec((TILE, 128), lambda i: (i, 0)),
        out_shape=jax.ShapeDtypeStruct((B * NPOINT, 128), jnp.float32),
    )(nx_pad, w1xpad)


def _stats1_body(g_ref, u_ref, sum_ref, sq_ref):
    i = pl.program_id(0)
    y1 = (g_ref[:, :].astype(jnp.float32).reshape(8, NSAMPLE, 128)
          - u_ref[:, :].reshape(8, 1, 128)).reshape(TILE, 128)
    ps = jnp.sum(y1.reshape(64, 8, 128), axis=0)
    pq = jnp.sum((y1 * y1).reshape(64, 8, 128), axis=0)

    @pl.when(i == 0)
    def _():
        sum_ref[:, :] = jnp.zeros_like(sum_ref)
        sq_ref[:, :] = jnp.zeros_like(sq_ref)

    sum_ref[:, :] += ps
    sq_ref[:, :] += pq


def _stats1(g, u):
    return pl.pallas_call(
        _stats1_body,
        grid=(GRID,),
        in_specs=[
            pl.BlockSpec((TILE, 128), lambda i: (i, 0)),
            pl.BlockSpec((8, 128), lambda i: (i, 0)),
        ],
        out_specs=[
            pl.BlockSpec((8, 128), lambda i: (0, 0)),
            pl.BlockSpec((8, 128), lambda i: (0, 0)),
        ],
        out_shape=[jax.ShapeDtypeStruct((8, 128), jnp.float32)] * 2,
    )(g, u)


def _affine(sum_, sq_, gamma, beta):
    mean = jnp.sum(sum_, axis=0, keepdims=True) / M
    var = jnp.sum(sq_, axis=0, keepdims=True) / M - mean * mean
    scale = gamma.reshape(1, -1) / jnp.sqrt(var + EPS)
    shift = beta.reshape(1, -1) - mean * scale
    return scale, shift


def _mlp2_body(g_ref, u_ref, sum_ref, sq_ref, ga_ref, be_ref, w2_ref,
               x1_ref, sum2_ref, sq2_ref):
    i = pl.program_id(0)
    scale, shift = _affine(sum_ref[:, :], sq_ref[:, :], ga_ref[:, :], be_ref[:, :])
    y1 = (g_ref[:, :].astype(jnp.float32).reshape(8, NSAMPLE, 128)
          - u_ref[:, :].reshape(8, 1, 128)).reshape(TILE, 128)
    x1 = jnp.maximum(y1 * scale + shift, 0.0).astype(jnp.bfloat16)
    x1_ref[:, :] = x1
    y2 = jnp.dot(x1, w2_ref[:, :], preferred_element_type=jnp.float32)
    ps = jnp.sum(y2.reshape(64, 8, 256), axis=0)
    pq = jnp.sum((y2 * y2).reshape(64, 8, 256), axis=0)

    @pl.when(i == 0)
    def _():
        sum2_ref[:, :] = jnp.zeros_like(sum2_ref)
        sq2_ref[:, :] = jnp.zeros_like(sq2_ref)

    sum2_ref[:, :] += ps
    sq2_ref[:, :] += pq


def _mlp2(g, u, s1, q1, g1, beta1, w2t):
    return pl.pallas_call(
        _mlp2_body,
        grid=(GRID,),
        in_specs=[
            pl.BlockSpec((TILE, 128), lambda i: (i, 0)),
            pl.BlockSpec((8, 128), lambda i: (i, 0)),
            pl.BlockSpec((8, 128), lambda i: (0, 0)),
            pl.BlockSpec((8, 128), lambda i: (0, 0)),
            pl.BlockSpec((1, 128), lambda i: (0, 0)),
            pl.BlockSpec((1, 128), lambda i: (0, 0)),
            pl.BlockSpec((128, 256), lambda i: (0, 0)),
        ],
        out_specs=[
            pl.BlockSpec((TILE, 128), lambda i: (i, 0)),
            pl.BlockSpec((8, 256), lambda i: (0, 0)),
            pl.BlockSpec((8, 256), lambda i: (0, 0)),
        ],
        out_shape=[
            jax.ShapeDtypeStruct((M, 128), jnp.bfloat16),
            jax.ShapeDtypeStruct((8, 256), jnp.float32),
            jax.ShapeDtypeStruct((8, 256), jnp.float32),
        ],
    )(g, u, s1, q1, g1.reshape(1, 128), beta1.reshape(1, 128), w2t)


def _mlp3_body(x1_ref, sum2_ref, sq2_ref, ga_ref, be_ref, w2_ref, w3_ref,
               mx_ref, mn_ref, sum3_ref, sq3_ref):
    i = pl.program_id(0)
    scale, shift = _affine(sum2_ref[:, :], sq2_ref[:, :], ga_ref[:, :], be_ref[:, :])
    y2 = jnp.dot(x1_ref[:, :], w2_ref[:, :], preferred_element_type=jnp.float32)
    x2 = jnp.maximum(y2 * scale + shift, 0.0).astype(jnp.bfloat16)
    y3 = jnp.dot(x2, w3_ref[:, :], preferred_element_type=jnp.float32)
    y3g = y3.reshape(8, NSAMPLE, 256)
    mx_ref[:, :] = jnp.max(y3g, axis=1)
    mn_ref[:, :] = jnp.min(y3g, axis=1)
    ps = jnp.sum(y3.reshape(64, 8, 256), axis=0)
    pq = jnp.sum((y3 * y3).reshape(64, 8, 256), axis=0)

    @pl.when(i == 0)
    def _():
        sum3_ref[:, :] = jnp.zeros_like(sum3_ref)
        sq3_ref[:, :] = jnp.zeros_like(sq3_ref)

    sum3_ref[:, :] += ps
    sq3_ref[:, :] += pq


def _mlp3(x1, s2, q2, g2, beta2, w2t, w3t):
    return pl.pallas_call(
        _mlp3_body,
        grid=(GRID,),
        in_specs=[
            pl.BlockSpec((TILE, 128), lambda i: (i, 0)),
            pl.BlockSpec((8, 256), lambda i: (0, 0)),
            pl.BlockSpec((8, 256), lambda i: (0, 0)),
            pl.BlockSpec((1, 256), lambda i: (0, 0)),
            pl.BlockSpec((1, 256), lambda i: (0, 0)),
            pl.BlockSpec((128, 256), lambda i: (0, 0)),
            pl.BlockSpec((256, 256), lambda i: (0, 0)),
        ],
        out_specs=[
            pl.BlockSpec((8, 256), lambda i: (i, 0)),
            pl.BlockSpec((8, 256), lambda i: (i, 0)),
            pl.BlockSpec((8, 256), lambda i: (0, 0)),
            pl.BlockSpec((8, 256), lambda i: (0, 0)),
        ],
        out_shape=[
            jax.ShapeDtypeStruct((B * NPOINT, 256), jnp.float32),
            jax.ShapeDtypeStruct((B * NPOINT, 256), jnp.float32),
            jax.ShapeDtypeStruct((8, 256), jnp.float32),
            jax.ShapeDtypeStruct((8, 256), jnp.float32),
        ],
    )(x1, s2, q2, g2.reshape(1, 256), beta2.reshape(1, 256), w2t, w3t)


def _fin_body(mx_ref, mn_ref, sum3_ref, sq3_ref, ga_ref, be_ref, o_ref):
    scale, shift = _affine(sum3_ref[:, :], sq3_ref[:, :], ga_ref[:, :], be_ref[:, :])
    hi = jnp.maximum(mx_ref[:, :] * scale + shift, 0.0)
    lo = jnp.maximum(mn_ref[:, :] * scale + shift, 0.0)
    o_ref[:, :] = jnp.where(scale > 0.0, hi, lo)


def _finalize(mx, mn, s3, q3, g3, beta3):
    return pl.pallas_call(
        _fin_body,
        grid=(B * NPOINT // TILE,),
        in_specs=[
            pl.BlockSpec((TILE, 256), lambda i: (i, 0)),
            pl.BlockSpec((TILE, 256), lambda i: (i, 0)),
            pl.BlockSpec((8, 256), lambda i: (0, 0)),
            pl.BlockSpec((8, 256), lambda i: (0, 0)),
            pl.BlockSpec((1, 256), lambda i: (0, 0)),
            pl.BlockSpec((1, 256), lambda i: (0, 0)),
        ],
        out_specs=pl.BlockSpec((TILE, 256), lambda i: (i, 0)),
        out_shape=jax.ShapeDtypeStruct((B * NPOINT, 256), jnp.float32),
    )(mx, mn, s3, q3, g3.reshape(1, 256), beta3.reshape(1, 256))


_ROWS = NPOINT // 4
_NV = N // 16


def _bq_row(p, xs_v, ys_v, zs_v, nx_v, ny_v, nz_v, idx_v, base):
    cx = jnp.full((16,), nx_v[pl.ds(p, 16)][0], jnp.float32)
    cy = jnp.full((16,), ny_v[pl.ds(p, 16)][0], jnp.float32)
    cz = jnp.full((16,), nz_v[pl.ds(p, 16)][0], jnp.float32)
    pad = jnp.full((16,), base, jnp.int32)
    idx_v[pl.ds(0, 16)] = pad
    idx_v[pl.ds(16, 16)] = pad
    idx_v[pl.ds(32, 16)] = pad
    idx_v[pl.ds(48, 16)] = pad
    lanes = lax.iota(jnp.int32, 16)

    def inner(i, cnt):
        off = pl.multiple_of(i * 16, 16)
        xv = xs_v[pl.ds(off, 16)]
        yv = ys_v[pl.ds(off, 16)]
        zv = zs_v[pl.ds(off, 16)]
        dx = xv - cx
        dy = yv - cy
        dz = zv - cz
        d2 = dx * dx + dy * dy + dz * dz
        m = d2 < R2
        vidx = lanes + (off + base)
        rank = plsc.cumsum(jnp.where(m, jnp.int32(1), jnp.int32(0)))
        plsc.store_scatter(idx_v, [cnt + rank - 1], vidx, mask=m)
        return cnt + plsc.all_reduce_population_count(m)

    cnt_vec = lax.fori_loop(0, _NV, inner, jnp.zeros((16,), jnp.int32))
    idx_v[pl.ds(cnt_vec[0], 16)] = pad


_NS = 4


_PROWS = 64


def _bq_gather_body(xs_h, ys_h, zs_h, nx_h, ny_h, nz_h, c1_h, g_h, *rest):
    xs_v, ys_v, zs_v, nx_v, ny_v, nz_v, idxw, spm = rest[:8]
    idx64 = rest[8:8 + _NS]
    gba = rest[8 + _NS:8 + 2 * _NS]
    gbb = rest[8 + 2 * _NS:8 + 3 * _NS]
    sga = rest[8 + 3 * _NS:8 + 4 * _NS]
    sgb = rest[8 + 4 * _NS:8 + 5 * _NS]
    soa = rest[8 + 5 * _NS:8 + 6 * _NS]
    sob = rest[8 + 6 * _NS:8 + 7 * _NS]
    cid = lax.axis_index("c")
    sid = lax.axis_index("s")

    def compute_into(p, s, base):
        _bq_row(p, xs_v, ys_v, zs_v, nx_v, ny_v, nz_v, idxw, base)
        for j in range(4):
            idx64[s][pl.ds(16 * j, 16)] = idxw[pl.ds(16 * j, 16)]

    def gather_start(s, gb, sg):
        pltpu.async_copy(spm.at[idx64[s]], gb[s], sg[s])

    def gather_wait(s, gb, sg):
        pltpu.make_async_copy(spm.at[idx64[s]], gb[s], sg[s]).wait()

    for pp in range(4):
        b = 4 * cid + pp
        pb = sid * _PROWS
        gp0 = b * NPOINT + pb
        base = 0
        pltpu.sync_copy(c1_h.at[pl.ds(b * N + sid * 256, 256)],
                        spm.at[pl.ds(sid * 256, 256)])
        pltpu.sync_copy(xs_h.at[b], xs_v)
        pltpu.sync_copy(ys_h.at[b], ys_v)
        pltpu.sync_copy(zs_h.at[b], zs_v)
        pltpu.sync_copy(nx_h.at[b, pl.ds(pb, _PROWS)], nx_v.at[pl.ds(0, _PROWS)])
        pltpu.sync_copy(ny_h.at[b, pl.ds(pb, _PROWS)], ny_v.at[pl.ds(0, _PROWS)])
        pltpu.sync_copy(nz_h.at[b, pl.ds(pb, _PROWS)], nz_v.at[pl.ds(0, _PROWS)])
        plsc.subcore_barrier()

        def out_start(r, s, gb, so):
            pltpu.async_copy(gb[s], g_h.at[gp0 + r], so[s])

        def out_wait(s, gb, so):
            pltpu.make_async_copy(gb[s], g_h.at[0], so[s]).wait()

        for s in range(_NS):
            compute_into(s, s, base)
            gather_start(s, gba, sga)
        for s in range(_NS):
            gather_wait(s, gba, sga)
            out_start(s, s, gba, soa)
            compute_into(_NS + s, s, base)
            gather_start(s, gbb, sgb)

        def pairbody(ii, _):
            r1 = _NS * (2 * ii + 1)
            for s in range(_NS):
                gather_wait(s, gbb, sgb)
                out_wait(s, gba, soa)
                out_start(r1 + s, s, gbb, sob)
                compute_into(r1 + _NS + s, s, base)
                gather_start(s, gba, sga)
            r2 = r1 + _NS
            for s in range(_NS):
                gather_wait(s, gba, sga)
                out_wait(s, gbb, sob)
                out_start(r2 + s, s, gba, soa)
                compute_into(r2 + _NS + s, s, base)
                gather_start(s, gbb, sgb)
            return 0

        lax.fori_loop(0, (_PROWS - 2 * _NS) // (2 * _NS), pairbody, 0)
        for s in range(_NS):
            gather_wait(s, gbb, sgb)
            out_wait(s, gba, soa)
            out_start(_PROWS - _NS + s, s, gbb, sob)
        for s in range(_NS):
            out_wait(s, gbb, sob)
        plsc.subcore_barrier()


def _bq_gather(xyz, new_xyz, c1):
    mesh = plsc.VectorSubcoreMesh(core_axis_name="c", subcore_axis_name="s")
    scratch = [
        pltpu.VMEM((N,), jnp.float32),
        pltpu.VMEM((N,), jnp.float32),
        pltpu.VMEM((N,), jnp.float32),
        pltpu.VMEM((_ROWS + 16,), jnp.float32),
        pltpu.VMEM((_ROWS + 16,), jnp.float32),
        pltpu.VMEM((_ROWS + 16,), jnp.float32),
        pltpu.VMEM((N + 16,), jnp.int32),
        pltpu.VMEM_SHARED((N, 128), jnp.float32),
    ]
    scratch += [pltpu.VMEM((NSAMPLE,), jnp.int32) for _ in range(_NS)]
    scratch += [pltpu.VMEM((NSAMPLE, 128), jnp.float32) for _ in range(2 * _NS)]
    scratch += [pltpu.SemaphoreType.DMA for _ in range(4 * _NS)]
    f = pl.kernel(
        _bq_gather_body,
        out_type=jax.ShapeDtypeStruct((B * NPOINT, NSAMPLE, 128), jnp.float32),
        mesh=mesh,
        compiler_params=pltpu.CompilerParams(needs_layout_passes=False),
        scratch_types=scratch,
    )
    return f(xyz[:, :, 0], xyz[:, :, 1], xyz[:, :, 2],
             new_xyz[:, :, 0], new_xyz[:, :, 1], new_xyz[:, :, 2],
             c1.reshape(B * N, 128))


def kernel(xyz, features, W1, b1, g1, beta1, W2, b2, g2, beta2, W3, b3, g3, beta3):
    fidx = _fps_pallas(xyz)
    new_xyz = jax.vmap(lambda p, i: p[i])(xyz, fidx)

    p_in = jnp.concatenate(
        [jnp.transpose(features, (0, 2, 1)), xyz,
         jnp.zeros((B, N, 5), jnp.float32)], axis=2)
    w1pad = jnp.concatenate(
        [jnp.transpose(W1[:, 3:131]), jnp.transpose(W1[:, 0:3]),
         jnp.zeros((5, 128), jnp.float32)], axis=0)
    c1 = _ptconv(p_in, w1pad, b1)

    nx_pad = jnp.concatenate(
        [new_xyz.reshape(B * NPOINT, 3),
         jnp.zeros((B * NPOINT, 5), jnp.float32)], axis=1)
    w1xpad = jnp.concatenate(
        [jnp.transpose(W1[:, 0:3]), jnp.zeros((5, 128), jnp.float32)], axis=0)
    u = _u_mat(nx_pad, w1xpad)

    g = _bq_gather(xyz, new_xyz, c1).reshape(M, 128)

    s1, q1 = _stats1(g, u)
    w2t = jnp.transpose(W2).astype(jnp.bfloat16)
    w3t = jnp.transpose(W3).astype(jnp.bfloat16)
    x1, s2, q2 = _mlp2(g, u, s1, q1, g1, beta1, w2t)
    mx, mn, s3, q3 = _mlp3(x1, s2, q2, g2, beta2, w2t, w3t)
    out = _finalize(mx, mn, s3, q3, g3, beta3)
    new_features = jnp.transpose(out.reshape(B, NPOINT, 256), (0, 2, 1))
    return new_xyz, new_features
